# baseline (speedup 1.0000x reference)
import functools
import jax
import jax.numpy as jnp
from jax import lax
from jax.experimental import pallas as pl
from jax.experimental.pallas import tpu as pltpu
from jax.experimental.pallas import tpu_sc as plsc


def _enc_body(x_ref, w_ref, b_ref, o_ref):
    o_ref[...] = jnp.maximum(
        jnp.dot(x_ref[...], w_ref[...], preferred_element_type=jnp.float32)
        + b_ref[...], 0.0)


def _tc_encode(x, W, b, blk):
    n, din = x.shape
    dout = W.shape[1]
    assert n % blk == 0
    return pl.pallas_call(
        _enc_body,
        grid=(n // blk,),
        in_specs=[
            pl.BlockSpec((blk, din), lambda i: (i, 0)),
            pl.BlockSpec((din, dout), lambda i: (0, 0)),
            pl.BlockSpec((1, dout), lambda i: (0, 0)),
        ],
        out_specs=pl.BlockSpec((blk, dout), lambda i: (i, 0)),
        out_shape=jax.ShapeDtypeStruct((n, dout), jnp.float32),
    )(x, W, b.reshape(1, dout))


def _proj_body(scale, h_ref, wq_ref, bq_ref, wk_ref, bk_ref, wv_ref, bv_ref,
               we_ref, bec_ref, qcat_ref, k_ref, v_ref):
    h = h_ref[...]
    q = jnp.dot(h, wq_ref[...], preferred_element_type=jnp.float32) + bq_ref[...]
    qs = q * scale
    qe = jnp.dot(qs, we_ref[...].T, preferred_element_type=jnp.float32)
    qcat_ref[...] = jnp.concatenate([qs, qe], axis=1)
    k_ref[...] = jnp.dot(h, wk_ref[...], preferred_element_type=jnp.float32) \
        + bk_ref[...] + bec_ref[...]
    v = jnp.dot(h, wv_ref[...], preferred_element_type=jnp.float32) + bv_ref[...]
    ones = jnp.ones((h.shape[0], 1), jnp.float32)
    vpad = jnp.zeros((h.shape[0], 15), jnp.float32)
    v_ref[...] = jnp.concatenate([v, ones, vpad], axis=1)


def _tc_proj(h, p, blk):
    n, d = h.shape
    scale = 1.0 / float(d) ** 0.5
    assert n % blk == 0
    full = lambda a, b: pl.BlockSpec((a, b), lambda i: (0, 0))
    return pl.pallas_call(
        functools.partial(_proj_body, scale),
        grid=(n // blk,),
        in_specs=[
            pl.BlockSpec((blk, d), lambda i: (i, 0)),
            full(d, d), full(1, d),
            full(d, d), full(1, d),
            full(d, d), full(1, d),
            full(d, d), full(1, d),
        ],
        out_specs=[
            pl.BlockSpec((blk, d + d), lambda i: (i, 0)),
            pl.BlockSpec((blk, d), lambda i: (i, 0)),
            pl.BlockSpec((blk, d + 16), lambda i: (i, 0)),
        ],
        out_shape=[
            jax.ShapeDtypeStruct((n, d + d), jnp.float32),
            jax.ShapeDtypeStruct((n, d), jnp.float32),
            jax.ShapeDtypeStruct((n, d + 16), jnp.float32),
        ],
    )(h, p["q"]["W"], p["q"]["b"].reshape(1, d),
      p["k"]["W"], p["k"]["b"].reshape(1, d),
      p["v"]["W"], p["v"]["b"].reshape(1, d),
      p["e"]["W"], p["e"]["b"].reshape(1, d))


def _epi_body(relu, aa_ref, ab_ref, ga_ref, gb_ref, h_ref,
              we_ref, be_ref, ws_ref, bs_ref, o_ref):
    d = h_ref.shape[1]
    acc = aa_ref[...] + ab_ref[...]
    av = acc[:, :d]
    den = acc[:, d:d + 1]
    g = ga_ref[...] + gb_ref[...]
    num = av + jnp.dot(g, we_ref[...], preferred_element_type=jnp.float32) \
        + den * be_ref[...]
    out = num / (den + 1e-16) \
        + jnp.dot(h_ref[...], ws_ref[...], preferred_element_type=jnp.float32) \
        + bs_ref[...]
    o_ref[...] = jnp.maximum(out, 0.0) if relu else out


def _tc_epilogue(accv2, g2, h, p, relu, blk):
    n, d = h.shape
    full = lambda a, b: pl.BlockSpec((a, b), lambda i: (0, 0))
    row = lambda c: pl.BlockSpec((blk, c), lambda i: (i, 0))
    return pl.pallas_call(
        functools.partial(_epi_body, relu),
        grid=(n // blk,),
        in_specs=[row(d + 16), row(d + 16), row(d), row(d), row(d),
                  full(d, d), full(1, d), full(d, d), full(1, d)],
        out_specs=row(d),
        out_shape=jax.ShapeDtypeStruct((n, d), jnp.float32),
    )(accv2[0], accv2[1], g2[0], g2[1], h,
      p["e"]["W"], p["e"]["b"].reshape(1, d),
      p["s"]["W"], p["s"]["b"].reshape(1, d))


def _lane_take(x, perm):
    dnums = lax.GatherDimensionNumbers(
        offset_dims=(), collapsed_slice_dims=(0,), start_index_map=(0,))
    return lax.gather(x, perm[:, None], dnums, slice_sizes=(1,),
                      mode=lax.GatherScatterMode.PROMISE_IN_BOUNDS)


_C = 80
_ZR = 25
_NEG = -1.0e30


def _sc_pass_a(dst, src, qcat, ktab, f):
    E = dst.shape[0]
    d = ktab.shape[1]
    dq = qcat.shape[1]
    info = plsc.get_sparse_core_info()
    nc, ns = info.num_cores, info.num_subcores
    nw = nc * ns
    assert E % (nw * _C) == 0
    ept = E // nw
    nchunk = ept // _C
    mesh = plsc.VectorSubcoreMesh(core_axis_name="c", subcore_axis_name="s")

    buf = lambda shape, dt=jnp.float32: [pltpu.VMEM(shape, dt),
                                         pltpu.VMEM(shape, dt)]

    @functools.partial(
        pl.kernel, mesh=mesh,
        compiler_params=pltpu.CompilerParams(needs_layout_passes=False, use_tc_tiling_on_sc=False),
        out_type=[jax.ShapeDtypeStruct((E,), jnp.float32),
                  jax.ShapeDtypeStruct((nw * 16,), jnp.float32)],
        scratch_types=[
            buf((_C,), jnp.int32), buf((_C,), jnp.int32),
            buf((_C, dq)), buf((_C, d)), buf((_C, d)),
            pltpu.VMEM((_C,), jnp.float32),
            pltpu.VMEM((16,), jnp.float32),
            [pltpu.SemaphoreType.DMA, pltpu.SemaphoreType.DMA],
            [pltpu.SemaphoreType.DMA, pltpu.SemaphoreType.DMA],
        ],
    )
    def body(dst_hbm, src_hbm, qcat_hbm, ktab_hbm, f_hbm,
             scores_hbm, tmax_hbm,
             dstv, srcv, qrows, krows, frows, scv, mx, sem_i, sem_g):
        wid = lax.axis_index("s") * nc + lax.axis_index("c")
        ebase = wid * ept
        mx[...] = jnp.full((16,), _NEG, jnp.float32)
        lane = lax.iota(jnp.int32, 16)
        lane0 = lane == 0
        perms = [lane ^ sh for sh in (8, 4, 2, 1)]

        def issue_ds(c, b):
            base = ebase + c * _C
            pltpu.async_copy(dst_hbm.at[pl.ds(base, _C)], dstv[b], sem_i[b])
            pltpu.async_copy(src_hbm.at[pl.ds(base, _C)], srcv[b], sem_i[b])

        def issue_f(c, b):
            base = ebase + c * _C
            pltpu.async_copy(f_hbm.at[pl.ds(base, _C)], frows[b], sem_i[b])

        def wait_idx(b):
            pltpu.make_async_copy(dst_hbm.at[pl.ds(0, _C)], dstv[b], sem_i[b]).wait()
            pltpu.make_async_copy(src_hbm.at[pl.ds(0, _C)], srcv[b], sem_i[b]).wait()
            pltpu.make_async_copy(f_hbm.at[pl.ds(0, _C)], frows[b], sem_i[b]).wait()

        def issue_gather(b):
            pltpu.async_copy(qcat_hbm.at[dstv[b]], qrows[b], sem_g[b])
            pltpu.async_copy(ktab_hbm.at[srcv[b]], krows[b], sem_g[b])

        def wait_gather(b):
            pltpu.make_async_copy(qcat_hbm.at[dstv[b]], qrows[b], sem_g[b]).wait()
            pltpu.make_async_copy(ktab_hbm.at[srcv[b]], krows[b], sem_g[b]).wait()

        def compute(c, b):
            qr, kr, fr = qrows[b], krows[b], frows[b]

            def quad(qi, _):
                for u in range(8):
                    j = qi * 8 + u
                    parts = []
                    for half in range(4):
                        a = qr[j, pl.ds(64 * half, 16)] * kf(kr, fr, j, 4 * half)
                        for t in range(4 * half + 1, 4 * half + 4):
                            a = a + qr[j, pl.ds(16 * t, 16)] * kf(kr, fr, j, t)
                        parts.append(a)
                    acc = (parts[0] + parts[1]) + (parts[2] + parts[3])
                    for perm in perms:
                        acc = acc + _lane_take(acc, perm)
                    plsc.store_scatter(scv, [jnp.full((16,), j, jnp.int32)],
                                       acc, mask=lane0)
                return 0

            lax.fori_loop(0, _C // 8, quad, 0)
            m = mx[...]
            for t in range(_C // 16):
                m = jnp.maximum(m, scv[pl.ds(16 * t, 16)])
            mx[...] = m
            pltpu.sync_copy(scv, scores_hbm.at[pl.ds(ebase + c * _C, _C)])

        def kf(kr, fr, j, t):
            if t < d // 16:
                return kr[j, pl.ds(16 * t, 16)]
            return fr[j, pl.ds(16 * (t - d // 16), 16)]

        issue_ds(0, 0)
        issue_f(0, 0)
        wait_idx(0)
        issue_gather(0)
        issue_ds(1, 1)
        issue_f(1, 1)

        def half(c, b, bn):
            @pl.when(c + 1 < nchunk)
            def _():
                wait_idx(bn)
                issue_gather(bn)

            @pl.when(c + 2 < nchunk)
            def _():
                issue_ds(c + 2, b)

            wait_gather(b)
            compute(c, b)

            @pl.when(c + 2 < nchunk)
            def _():
                issue_f(c + 2, b)

        def it(c, _):
            pl.when(c % 2 == 0)(lambda: half(c, 0, 1))
            pl.when(c % 2 == 1)(lambda: half(c, 1, 0))
            return 0

        lax.fori_loop(0, nchunk, it, 0)
        pltpu.sync_copy(mx, tmax_hbm.at[pl.ds(wid * 16, 16)])

    return body(dst, src, qcat, ktab, f)


def _sc_pass_b(dst, src, scores, tmax, tab, n_nodes):
    E = dst.shape[0]
    d = tab.shape[1]
    info = plsc.get_sparse_core_info()
    nc, ns = info.num_cores, info.num_subcores
    nw = nc * ns
    assert E % (nw * _C) == 0
    ept = E // nw
    nchunk = ept // _C
    rpt = n_nodes // ns
    assert n_nodes % ns == 0 and rpt % _ZR == 0
    gather = src is not None
    mesh = plsc.VectorSubcoreMesh(core_axis_name="c", subcore_axis_name="s")

    buf = lambda shape, dt=jnp.float32: [pltpu.VMEM(shape, dt),
                                         pltpu.VMEM(shape, dt)]
    sem2 = lambda: [pltpu.SemaphoreType.DMA, pltpu.SemaphoreType.DMA]
    out_type = jax.ShapeDtypeStruct((nc, n_nodes, d), jnp.float32)
    scratch = [
        buf((_C,), jnp.int32),
        buf((_C,), jnp.int32),
        buf((_C,), jnp.int32),
        buf((_C, d)),
        buf((_C, d)) if not gather else [pltpu.VMEM((_C, d), jnp.float32)],
        buf((_C,)),
        pltpu.VMEM((nw * 16,), jnp.float32),
        pltpu.VMEM((_ZR, d), jnp.float32),
        pltpu.MemorySpace.VMEM_SHARED((n_nodes, d), jnp.float32),
        sem2(), sem2(), sem2(),
    ]
    @functools.partial(pl.kernel, mesh=mesh, out_type=out_type,
                       compiler_params=pltpu.CompilerParams(
                           needs_layout_passes=False,
                           use_tc_tiling_on_sc=False),
                       scratch_types=scratch)
    def body(*refs):
        (dst_hbm, src_hbm, scores_hbm, tmax_hbm, tab_hbm, out_hbm,
         dstv, srcv, dsts, trows, srows, scv, tbuf, zbuf, acc,
         sem_i, sem_g, sem_s) = refs
        cid = lax.axis_index("c")
        sid = lax.axis_index("s")
        wid = sid * nc + cid
        ebase = wid * ept

        def zr(i, _):
            for t in range(d // 16):
                zbuf[i, pl.ds(16 * t, 16)] = jnp.zeros((16,), jnp.float32)
            return 0

        lax.fori_loop(0, _ZR, zr, 0)
        for b in range(rpt // _ZR):
            pltpu.sync_copy(zbuf, acc.at[pl.ds(sid * rpt + b * _ZR, _ZR)])

        pltpu.sync_copy(tmax_hbm, tbuf)
        mv = tbuf[pl.ds(0, 16)]
        for t in range(1, nw):
            mv = jnp.maximum(mv, tbuf[pl.ds(16 * t, 16)])

        def issue_idx(c, b):
            base = ebase + c * _C
            pltpu.async_copy(dst_hbm.at[pl.ds(base, _C)], dstv[b], sem_i[b])
            pltpu.async_copy(scores_hbm.at[pl.ds(base, _C)], scv[b], sem_i[b])
            if gather:
                pltpu.async_copy(src_hbm.at[pl.ds(base, _C)], srcv[b], sem_i[b])

        def wait_idx(b):
            pltpu.make_async_copy(dst_hbm.at[pl.ds(0, _C)], dstv[b], sem_i[b]).wait()
            pltpu.make_async_copy(scores_hbm.at[pl.ds(0, _C)], scv[b], sem_i[b]).wait()
            if gather:
                pltpu.make_async_copy(src_hbm.at[pl.ds(0, _C)], srcv[b], sem_i[b]).wait()

        def issue_rows(c, b):
            if gather:
                pltpu.async_copy(tab_hbm.at[srcv[b]], trows[b % len(trows)], sem_g[b])
            else:
                base = ebase + c * _C
                pltpu.async_copy(tab_hbm.at[pl.ds(base, _C)], trows[b % len(trows)], sem_g[b])

        def wait_rows(b):
            if gather:
                pltpu.make_async_copy(tab_hbm.at[srcv[b]], trows[b % len(trows)], sem_g[b]).wait()
            else:
                pltpu.make_async_copy(tab_hbm.at[pl.ds(0, _C)], trows[b % len(trows)], sem_g[b]).wait()

        def issue_scat(b):
            if gather:
                pltpu.sync_copy(srows[0], acc.at[dsts[b]], add=True)
            else:
                pltpu.async_copy(srows[b % len(srows)], acc.at[dsts[b]],
                                 sem_s[b], add=True)

        def wait_scat(b):
            if not gather:
                pltpu.make_async_copy(srows[b], acc.at[dsts[b]],
                                      sem_s[b]).wait()

        def compute(b):
            for t in range(_C // 16):
                dsts[b][pl.ds(16 * t, 16)] = dstv[b][pl.ds(16 * t, 16)]

            sr = srows[b % len(srows)]
            tr = trows[b % len(trows)]

            if gather:
                def grpw(g, _):
                    wgrp = jnp.exp(scv[b][pl.ds(g * 16, 16)] - mv)
                    for jj in range(16):
                        sr[g * 16 + jj, pl.ds(d - 16, 16)] = \
                            jnp.full((16,), wgrp[jj], jnp.float32)
                    return 0

                lax.fori_loop(0, _C // 16, grpw, 0)

                def edge(j, _):
                    wsvec = sr[j, pl.ds(d - 16, 16)]
                    for t in range(d // 16 - 1):
                        sr[j, pl.ds(16 * t, 16)] = \
                            wsvec * tr[j, pl.ds(16 * t, 16)]
                    return 0

                lax.fori_loop(0, _C, edge, 0)
            else:
                def grp(g, _):
                    wgrp = jnp.exp(scv[b][pl.ds(g * 16, 16)] - mv)
                    for jj in range(16):
                        j = g * 16 + jj
                        ws = wgrp[jj]
                        for t in range(d // 16):
                            sr[j, pl.ds(16 * t, 16)] = \
                                ws * tr[j, pl.ds(16 * t, 16)]
                    return 0

                lax.fori_loop(0, _C // 16, grp, 0)

        plsc.subcore_barrier()

        issue_idx(0, 0)
        wait_idx(0)
        issue_rows(0, 0)
        issue_idx(1, 1)

        def half(c, b, bn):
            @pl.when(c > 0)
            def _():
                wait_scat(bn)

            @pl.when(c + 1 < nchunk)
            def _():
                wait_idx(bn)
                issue_rows(c + 1, bn)

            wait_rows(b)
            compute(b)

            @pl.when(c + 2 < nchunk)
            def _():
                issue_idx(c + 2, b)

            issue_scat(b)

        def it(c, _):
            pl.when(c % 2 == 0)(lambda: half(c, 0, 1))
            pl.when(c % 2 == 1)(lambda: half(c, 1, 0))
            return 0

        lax.fori_loop(0, nchunk, it, 0)
        wait_scat((nchunk - 1) % 2)

        plsc.subcore_barrier()
        for b in range(rpt // _ZR):
            r0 = sid * rpt + b * _ZR
            pltpu.sync_copy(acc.at[pl.ds(r0, _ZR)], zbuf)
            pltpu.sync_copy(zbuf, out_hbm.at[cid, pl.ds(r0, _ZR)])

    if not gather:
        src = dst
    return body(dst, src, scores, tmax, tab)


def kernel(x, edge_index, edge_attr, params):
    N, _ = x.shape
    src = edge_index[0]
    dst = edge_index[1]
    p = params

    f = _tc_encode(edge_attr, p["enc_e"]["W"], p["enc_e"]["b"], blk=4000)
    h = _tc_encode(x, p["enc_n"]["W"], p["enc_n"]["b"], blk=1000)
    d = h.shape[1]

    for li, layer in enumerate(("c1", "c2", "c3")):
        lp = p[layer]
        qcat, ktab, vtab = _tc_proj(h, lp, blk=1000)
        scores, tmax = _sc_pass_a(dst, src, qcat, ktab, f)
        accv2 = _sc_pass_b(dst, src, scores, tmax, vtab, N)
        g2 = _sc_pass_b(dst, None, scores, tmax, f, N)
        h = _tc_epilogue(accv2, g2, h, lp, relu=(li < 2), blk=1000)
    return h


# speedup vs baseline: 1.0092x; 1.0092x over previous
import functools
import jax
import jax.numpy as jnp
from jax import lax
from jax.experimental import pallas as pl
from jax.experimental.pallas import tpu as pltpu
from jax.experimental.pallas import tpu_sc as plsc


def _enc_body(x_ref, w_ref, b_ref, o_ref):
    o_ref[...] = jnp.maximum(
        jnp.dot(x_ref[...], w_ref[...], preferred_element_type=jnp.float32)
        + b_ref[...], 0.0)


def _tc_encode(x, W, b, blk):
    n, din = x.shape
    dout = W.shape[1]
    assert n % blk == 0
    return pl.pallas_call(
        _enc_body,
        grid=(n // blk,),
        in_specs=[
            pl.BlockSpec((blk, din), lambda i: (i, 0)),
            pl.BlockSpec((din, dout), lambda i: (0, 0)),
            pl.BlockSpec((1, dout), lambda i: (0, 0)),
        ],
        out_specs=pl.BlockSpec((blk, dout), lambda i: (i, 0)),
        out_shape=jax.ShapeDtypeStruct((n, dout), jnp.float32),
    )(x, W, b.reshape(1, dout))


def _proj_body(scale, h_ref, wq_ref, bq_ref, wk_ref, bk_ref, wv_ref, bv_ref,
               we_ref, bec_ref, qcat_ref, k_ref, v_ref):
    h = h_ref[...]
    q = jnp.dot(h, wq_ref[...], preferred_element_type=jnp.float32) + bq_ref[...]
    qs = q * scale
    qe = jnp.dot(qs, we_ref[...].T, preferred_element_type=jnp.float32)
    qcat_ref[...] = jnp.concatenate([qs, qe], axis=1)
    k_ref[...] = jnp.dot(h, wk_ref[...], preferred_element_type=jnp.float32) \
        + bk_ref[...] + bec_ref[...]
    v = jnp.dot(h, wv_ref[...], preferred_element_type=jnp.float32) + bv_ref[...]
    ones = jnp.ones((h.shape[0], 1), jnp.float32)
    vpad = jnp.zeros((h.shape[0], 15), jnp.float32)
    v_ref[...] = jnp.concatenate([v, ones, vpad], axis=1)


def _tc_proj(h, p, blk):
    n, d = h.shape
    scale = 1.0 / float(d) ** 0.5
    assert n % blk == 0
    full = lambda a, b: pl.BlockSpec((a, b), lambda i: (0, 0))
    return pl.pallas_call(
        functools.partial(_proj_body, scale),
        grid=(n // blk,),
        in_specs=[
            pl.BlockSpec((blk, d), lambda i: (i, 0)),
            full(d, d), full(1, d),
            full(d, d), full(1, d),
            full(d, d), full(1, d),
            full(d, d), full(1, d),
        ],
        out_specs=[
            pl.BlockSpec((blk, d + d), lambda i: (i, 0)),
            pl.BlockSpec((blk, d), lambda i: (i, 0)),
            pl.BlockSpec((blk, d + 16), lambda i: (i, 0)),
        ],
        out_shape=[
            jax.ShapeDtypeStruct((n, d + d), jnp.float32),
            jax.ShapeDtypeStruct((n, d), jnp.float32),
            jax.ShapeDtypeStruct((n, d + 16), jnp.float32),
        ],
    )(h, p["q"]["W"], p["q"]["b"].reshape(1, d),
      p["k"]["W"], p["k"]["b"].reshape(1, d),
      p["v"]["W"], p["v"]["b"].reshape(1, d),
      p["e"]["W"], p["e"]["b"].reshape(1, d))


def _epi_body(relu, aa_ref, ab_ref, ga_ref, gb_ref, h_ref,
              we_ref, be_ref, ws_ref, bs_ref, o_ref):
    d = h_ref.shape[1]
    acc = aa_ref[...] + ab_ref[...]
    av = acc[:, :d]
    den = acc[:, d:d + 1]
    g = ga_ref[...] + gb_ref[...]
    num = av + jnp.dot(g, we_ref[...], preferred_element_type=jnp.float32) \
        + den * be_ref[...]
    out = num / (den + 1e-16) \
        + jnp.dot(h_ref[...], ws_ref[...], preferred_element_type=jnp.float32) \
        + bs_ref[...]
    o_ref[...] = jnp.maximum(out, 0.0) if relu else out


def _tc_epilogue(accv2, g2, h, p, relu, blk):
    n, d = h.shape
    full = lambda a, b: pl.BlockSpec((a, b), lambda i: (0, 0))
    row = lambda c: pl.BlockSpec((blk, c), lambda i: (i, 0))
    return pl.pallas_call(
        functools.partial(_epi_body, relu),
        grid=(n // blk,),
        in_specs=[row(d + 16), row(d + 16), row(d), row(d), row(d),
                  full(d, d), full(1, d), full(d, d), full(1, d)],
        out_specs=row(d),
        out_shape=jax.ShapeDtypeStruct((n, d), jnp.float32),
    )(accv2[0], accv2[1], g2[0], g2[1], h,
      p["e"]["W"], p["e"]["b"].reshape(1, d),
      p["s"]["W"], p["s"]["b"].reshape(1, d))


def _lane_take(x, perm):
    dnums = lax.GatherDimensionNumbers(
        offset_dims=(), collapsed_slice_dims=(0,), start_index_map=(0,))
    return lax.gather(x, perm[:, None], dnums, slice_sizes=(1,),
                      mode=lax.GatherScatterMode.PROMISE_IN_BOUNDS)


_C = 80
_ZR = 25
_NEG = -1.0e30


def _sc_pass_a(dst, src, qcat, ktab, f):
    E = dst.shape[0]
    d = ktab.shape[1]
    dq = qcat.shape[1]
    info = plsc.get_sparse_core_info()
    nc, ns = info.num_cores, info.num_subcores
    nw = nc * ns
    assert E % (nw * _C) == 0
    ept = E // nw
    nchunk = ept // _C
    mesh = plsc.VectorSubcoreMesh(core_axis_name="c", subcore_axis_name="s")

    buf = lambda shape, dt=jnp.float32: [pltpu.VMEM(shape, dt),
                                         pltpu.VMEM(shape, dt)]

    @functools.partial(
        pl.kernel, mesh=mesh,
        compiler_params=pltpu.CompilerParams(needs_layout_passes=False, use_tc_tiling_on_sc=False),
        out_type=[jax.ShapeDtypeStruct((E,), jnp.float32),
                  jax.ShapeDtypeStruct((nw * 16,), jnp.float32)],
        scratch_types=[
            buf((_C,), jnp.int32), buf((_C,), jnp.int32),
            buf((_C, dq)), buf((_C, d)), buf((_C, d)),
            buf((_C,)),
            pltpu.VMEM((16,), jnp.float32),
            [pltpu.SemaphoreType.DMA, pltpu.SemaphoreType.DMA],
            [pltpu.SemaphoreType.DMA, pltpu.SemaphoreType.DMA],
            [pltpu.SemaphoreType.DMA, pltpu.SemaphoreType.DMA],
        ],
    )
    def body(dst_hbm, src_hbm, qcat_hbm, ktab_hbm, f_hbm,
             scores_hbm, tmax_hbm,
             dstv, srcv, qrows, krows, frows, scv, mx, sem_i, sem_g, sem_o):
        wid = lax.axis_index("s") * nc + lax.axis_index("c")
        ebase = wid * ept
        mx[...] = jnp.full((16,), _NEG, jnp.float32)
        lane = lax.iota(jnp.int32, 16)
        lane0 = lane == 0
        perms = [lane ^ sh for sh in (8, 4, 2, 1)]

        def issue_ds(c, b):
            base = ebase + c * _C
            pltpu.async_copy(dst_hbm.at[pl.ds(base, _C)], dstv[b], sem_i[b])
            pltpu.async_copy(src_hbm.at[pl.ds(base, _C)], srcv[b], sem_i[b])

        def issue_f(c, b):
            base = ebase + c * _C
            pltpu.async_copy(f_hbm.at[pl.ds(base, _C)], frows[b], sem_i[b])

        def wait_idx(b):
            pltpu.make_async_copy(dst_hbm.at[pl.ds(0, _C)], dstv[b], sem_i[b]).wait()
            pltpu.make_async_copy(src_hbm.at[pl.ds(0, _C)], srcv[b], sem_i[b]).wait()
            pltpu.make_async_copy(f_hbm.at[pl.ds(0, _C)], frows[b], sem_i[b]).wait()

        def issue_gather(b):
            pltpu.async_copy(qcat_hbm.at[dstv[b]], qrows[b], sem_g[b])
            pltpu.async_copy(ktab_hbm.at[srcv[b]], krows[b], sem_g[b])

        def wait_gather(b):
            pltpu.make_async_copy(qcat_hbm.at[dstv[b]], qrows[b], sem_g[b]).wait()
            pltpu.make_async_copy(ktab_hbm.at[srcv[b]], krows[b], sem_g[b]).wait()

        def compute(c, b):
            qr, kr, fr = qrows[b], krows[b], frows[b]

            def quad(qi, _):
                for u in range(4):
                    j = qi * 4 + u
                    parts = []
                    for half in range(4):
                        a = qr[j, pl.ds(64 * half, 16)] * kf(kr, fr, j, 4 * half)
                        for t in range(4 * half + 1, 4 * half + 4):
                            a = a + qr[j, pl.ds(16 * t, 16)] * kf(kr, fr, j, t)
                        parts.append(a)
                    acc = (parts[0] + parts[1]) + (parts[2] + parts[3])
                    for perm in perms:
                        acc = acc + _lane_take(acc, perm)
                    plsc.store_scatter(scv[b], [jnp.full((16,), j, jnp.int32)],
                                       acc, mask=lane0)
                return 0

            lax.fori_loop(0, _C // 4, quad, 0)
            m = mx[...]
            for t in range(_C // 16):
                m = jnp.maximum(m, scv[b][pl.ds(16 * t, 16)])
            mx[...] = m
            pltpu.async_copy(scv[b], scores_hbm.at[pl.ds(ebase + c * _C, _C)],
                             sem_o[b])

        def kf(kr, fr, j, t):
            if t < d // 16:
                return kr[j, pl.ds(16 * t, 16)]
            return fr[j, pl.ds(16 * (t - d // 16), 16)]

        issue_ds(0, 0)
        issue_f(0, 0)
        wait_idx(0)
        issue_gather(0)
        issue_ds(1, 1)
        issue_f(1, 1)

        def half(c, b, bn):
            @pl.when(c + 1 < nchunk)
            def _():
                wait_idx(bn)
                issue_gather(bn)

            @pl.when(c + 2 < nchunk)
            def _():
                issue_ds(c + 2, b)

            wait_gather(b)

            @pl.when(c > 1)
            def _():
                pltpu.make_async_copy(
                    scv[b], scores_hbm.at[pl.ds(0, _C)], sem_o[b]).wait()

            compute(c, b)

            @pl.when(c + 2 < nchunk)
            def _():
                issue_f(c + 2, b)

        def it(c, _):
            pl.when(c % 2 == 0)(lambda: half(c, 0, 1))
            pl.when(c % 2 == 1)(lambda: half(c, 1, 0))
            return 0

        lax.fori_loop(0, nchunk, it, 0)
        pltpu.make_async_copy(scv[(nchunk - 1) % 2],
                              scores_hbm.at[pl.ds(0, _C)], sem_o[(nchunk - 1) % 2]).wait()
        pltpu.make_async_copy(scv[(nchunk - 2) % 2],
                              scores_hbm.at[pl.ds(0, _C)], sem_o[(nchunk - 2) % 2]).wait()
        pltpu.sync_copy(mx, tmax_hbm.at[pl.ds(wid * 16, 16)])

    return body(dst, src, qcat, ktab, f)


def _sc_pass_b(dst, src, scores, tmax, tab, n_nodes):
    E = dst.shape[0]
    d = tab.shape[1]
    info = plsc.get_sparse_core_info()
    nc, ns = info.num_cores, info.num_subcores
    nw = nc * ns
    assert E % (nw * _C) == 0
    ept = E // nw
    nchunk = ept // _C
    rpt = n_nodes // ns
    assert n_nodes % ns == 0 and rpt % _ZR == 0
    gather = src is not None
    mesh = plsc.VectorSubcoreMesh(core_axis_name="c", subcore_axis_name="s")

    buf = lambda shape, dt=jnp.float32: [pltpu.VMEM(shape, dt),
                                         pltpu.VMEM(shape, dt)]
    sem2 = lambda: [pltpu.SemaphoreType.DMA, pltpu.SemaphoreType.DMA]
    out_type = jax.ShapeDtypeStruct((nc, n_nodes, d), jnp.float32)
    scratch = [
        buf((_C,), jnp.int32),
        buf((_C,), jnp.int32),
        buf((_C,), jnp.int32),
        buf((_C, d)),
        buf((_C, d)) if not gather else [pltpu.VMEM((_C, d), jnp.float32)],
        buf((_C,)),
        pltpu.VMEM((nw * 16,), jnp.float32),
        pltpu.VMEM((_ZR, d), jnp.float32),
        pltpu.MemorySpace.VMEM_SHARED((n_nodes, d), jnp.float32),
        sem2(), sem2(), sem2(),
    ]
    @functools.partial(pl.kernel, mesh=mesh, out_type=out_type,
                       compiler_params=pltpu.CompilerParams(
                           needs_layout_passes=False,
                           use_tc_tiling_on_sc=False),
                       scratch_types=scratch)
    def body(*refs):
        (dst_hbm, src_hbm, scores_hbm, tmax_hbm, tab_hbm, out_hbm,
         dstv, srcv, dsts, trows, srows, scv, tbuf, zbuf, acc,
         sem_i, sem_g, sem_s) = refs
        cid = lax.axis_index("c")
        sid = lax.axis_index("s")
        wid = sid * nc + cid
        ebase = wid * ept

        def zr(i, _):
            for t in range(d // 16):
                zbuf[i, pl.ds(16 * t, 16)] = jnp.zeros((16,), jnp.float32)
            return 0

        lax.fori_loop(0, _ZR, zr, 0)
        for b in range(rpt // _ZR):
            pltpu.sync_copy(zbuf, acc.at[pl.ds(sid * rpt + b * _ZR, _ZR)])

        pltpu.sync_copy(tmax_hbm, tbuf)
        mv = tbuf[pl.ds(0, 16)]
        for t in range(1, nw):
            mv = jnp.maximum(mv, tbuf[pl.ds(16 * t, 16)])

        def issue_idx(c, b):
            base = ebase + c * _C
            pltpu.async_copy(dst_hbm.at[pl.ds(base, _C)], dstv[b], sem_i[b])
            pltpu.async_copy(scores_hbm.at[pl.ds(base, _C)], scv[b], sem_i[b])
            if gather:
                pltpu.async_copy(src_hbm.at[pl.ds(base, _C)], srcv[b], sem_i[b])

        def wait_idx(b):
            pltpu.make_async_copy(dst_hbm.at[pl.ds(0, _C)], dstv[b], sem_i[b]).wait()
            pltpu.make_async_copy(scores_hbm.at[pl.ds(0, _C)], scv[b], sem_i[b]).wait()
            if gather:
                pltpu.make_async_copy(src_hbm.at[pl.ds(0, _C)], srcv[b], sem_i[b]).wait()

        def issue_rows(c, b):
            if gather:
                pltpu.async_copy(tab_hbm.at[srcv[b]], trows[b % len(trows)], sem_g[b])
            else:
                base = ebase + c * _C
                pltpu.async_copy(tab_hbm.at[pl.ds(base, _C)], trows[b % len(trows)], sem_g[b])

        def wait_rows(b):
            if gather:
                pltpu.make_async_copy(tab_hbm.at[srcv[b]], trows[b % len(trows)], sem_g[b]).wait()
            else:
                pltpu.make_async_copy(tab_hbm.at[pl.ds(0, _C)], trows[b % len(trows)], sem_g[b]).wait()

        def issue_scat(b):
            if gather:
                pltpu.sync_copy(srows[0], acc.at[dsts[b]], add=True)
            else:
                pltpu.async_copy(srows[b % len(srows)], acc.at[dsts[b]],
                                 sem_s[b], add=True)

        def wait_scat(b):
            if not gather:
                pltpu.make_async_copy(srows[b], acc.at[dsts[b]],
                                      sem_s[b]).wait()

        def compute(b):
            for t in range(_C // 16):
                dsts[b][pl.ds(16 * t, 16)] = dstv[b][pl.ds(16 * t, 16)]

            sr = srows[b % len(srows)]
            tr = trows[b % len(trows)]

            if gather:
                def grpw(g, _):
                    wgrp = jnp.exp(scv[b][pl.ds(g * 16, 16)] - mv)
                    for jj in range(16):
                        sr[g * 16 + jj, pl.ds(d - 16, 16)] = \
                            jnp.full((16,), wgrp[jj], jnp.float32)
                    return 0

                lax.fori_loop(0, _C // 16, grpw, 0)

                def edge(j, _):
                    wsvec = sr[j, pl.ds(d - 16, 16)]
                    for t in range(d // 16 - 1):
                        sr[j, pl.ds(16 * t, 16)] = \
                            wsvec * tr[j, pl.ds(16 * t, 16)]
                    return 0

                lax.fori_loop(0, _C, edge, 0)
            else:
                def grp(g, _):
                    wgrp = jnp.exp(scv[b][pl.ds(g * 16, 16)] - mv)
                    for jj in range(16):
                        j = g * 16 + jj
                        ws = wgrp[jj]
                        for t in range(d // 16):
                            sr[j, pl.ds(16 * t, 16)] = \
                                ws * tr[j, pl.ds(16 * t, 16)]
                    return 0

                lax.fori_loop(0, _C // 16, grp, 0)

        plsc.subcore_barrier()

        issue_idx(0, 0)
        wait_idx(0)
        issue_rows(0, 0)
        issue_idx(1, 1)

        def half(c, b, bn):
            @pl.when(c > 0)
            def _():
                wait_scat(bn)

            @pl.when(c + 1 < nchunk)
            def _():
                wait_idx(bn)
                issue_rows(c + 1, bn)

            wait_rows(b)
            compute(b)

            @pl.when(c + 2 < nchunk)
            def _():
                issue_idx(c + 2, b)

            issue_scat(b)

        def it(c, _):
            pl.when(c % 2 == 0)(lambda: half(c, 0, 1))
            pl.when(c % 2 == 1)(lambda: half(c, 1, 0))
            return 0

        lax.fori_loop(0, nchunk, it, 0)
        wait_scat((nchunk - 1) % 2)

        plsc.subcore_barrier()
        for b in range(rpt // _ZR):
            r0 = sid * rpt + b * _ZR
            pltpu.sync_copy(acc.at[pl.ds(r0, _ZR)], zbuf)
            pltpu.sync_copy(zbuf, out_hbm.at[cid, pl.ds(r0, _ZR)])

    if not gather:
        src = dst
    return body(dst, src, scores, tmax, tab)


def kernel(x, edge_index, edge_attr, params):
    N, _ = x.shape
    src = edge_index[0]
    dst = edge_index[1]
    p = params

    f = _tc_encode(edge_attr, p["enc_e"]["W"], p["enc_e"]["b"], blk=4000)
    h = _tc_encode(x, p["enc_n"]["W"], p["enc_n"]["b"], blk=1000)
    d = h.shape[1]

    for li, layer in enumerate(("c1", "c2", "c3")):
        lp = p[layer]
        qcat, ktab, vtab = _tc_proj(h, lp, blk=1000)
        scores, tmax = _sc_pass_a(dst, src, qcat, ktab, f)
        accv2 = _sc_pass_b(dst, src, scores, tmax, vtab, N)
        g2 = _sc_pass_b(dst, None, scores, tmax, f, N)
        h = _tc_epilogue(accv2, g2, h, lp, relu=(li < 2), blk=1000)
    return h
